# baseline (speedup 1.0000x reference)
import jax
import jax.numpy as jnp
from jax.experimental import pallas as pl
from jax.experimental.pallas import tpu as pltpu

_TILE_M = 512
_CORES = 2


def _block_kernel(xl_ref, xr_ref, w0_ref, b0_ref, w1_ref, b1_ref, ws_ref,
                  o_ref, w0b_ref, w1b_ref, wsb_ref):
    j = pl.program_id(1)

    @pl.when(j == 0)
    def _cast_weights():
        w0b_ref[...] = w0_ref[...].astype(jnp.bfloat16)
        w1b_ref[...] = w1_ref[...].astype(jnp.bfloat16)
        wsb_ref[...] = ws_ref[...].astype(jnp.bfloat16)

    xb = jnp.concatenate(
        [xl_ref[...].astype(jnp.bfloat16), xr_ref[...].astype(jnp.bfloat16)],
        axis=1)
    h = jnp.maximum(xb, jnp.bfloat16(0))
    net = jnp.dot(h, w0b_ref[...], preferred_element_type=jnp.float32)
    net = jnp.maximum(net + b0_ref[...], 0.0).astype(jnp.bfloat16)
    acc = jnp.dot(xb, wsb_ref[...], preferred_element_type=jnp.float32)
    acc = acc + jnp.dot(net, w1b_ref[...], preferred_element_type=jnp.float32)
    o_ref[...] = acc + b1_ref[...]


@jax.jit
def _run(x, w0, b0, w1, b1, ws):
    n, size_in = x.shape
    size_h = w0.shape[1]
    size_out = w1.shape[1]
    half = size_in // 2
    n_steps = n // (_CORES * _TILE_M)

    out = pl.pallas_call(
        _block_kernel,
        out_shape=jax.ShapeDtypeStruct((n, size_out), jnp.float32),
        grid=(_CORES, n_steps),
        in_specs=[
            pl.BlockSpec((_TILE_M, half), lambda i, j: (i * n_steps + j, 0)),
            pl.BlockSpec((_TILE_M, half), lambda i, j: (i * n_steps + j, 1)),
            pl.BlockSpec((size_in, size_h), lambda i, j: (0, 0)),
            pl.BlockSpec((1, size_h), lambda i, j: (0, 0)),
            pl.BlockSpec((size_h, size_out), lambda i, j: (0, 0)),
            pl.BlockSpec((1, size_out), lambda i, j: (0, 0)),
            pl.BlockSpec((size_in, size_out), lambda i, j: (0, 0)),
        ],
        out_specs=pl.BlockSpec((_TILE_M, size_out),
                               lambda i, j: (i * n_steps + j, 0)),
        scratch_shapes=[
            pltpu.VMEM((size_in, size_h), jnp.bfloat16),
            pltpu.VMEM((size_h, size_out), jnp.bfloat16),
            pltpu.VMEM((size_in, size_out), jnp.bfloat16),
        ],
        compiler_params=pltpu.CompilerParams(
            dimension_semantics=("parallel", "arbitrary"),
            vmem_limit_bytes=60 * 1024 * 1024,
        ),
        cost_estimate=pl.CostEstimate(
            flops=2 * n * (size_in * size_h + size_h * size_out
                           + size_in * size_out),
            transcendentals=0,
            bytes_accessed=(4 * n * (size_in + size_out)
                            + 4 * (size_in * size_h + size_h * size_out
                                   + size_in * size_out)),
        ),
    )(x, x, w0, b0, w1, b1, ws)
    return out


def kernel(x, w0, b0, w1, b1, ws):
    return _run(x, w0, b0, w1, b1, ws)


# speedup vs baseline: 1.0121x; 1.0121x over previous
import jax
import jax.numpy as jnp
from jax.experimental import pallas as pl
from jax.experimental.pallas import tpu as pltpu

_TILE_M = 512


def _block_kernel(x_ref, w0_ref, b0_ref, w1_ref, b1_ref, ws_ref, o_ref,
                  w0b_ref, w1b_ref, wsb_ref):
    j = pl.program_id(1)

    @pl.when(j == 0)
    def _cast_weights():
        w0b_ref[...] = w0_ref[...].astype(jnp.bfloat16)
        w1b_ref[...] = w1_ref[...].astype(jnp.bfloat16)
        wsb_ref[...] = ws_ref[...].astype(jnp.bfloat16)

    xb = x_ref[...].astype(jnp.bfloat16)
    h = jnp.maximum(xb, jnp.bfloat16(0))
    net = jnp.dot(h, w0b_ref[...], preferred_element_type=jnp.float32)
    net = jnp.maximum(net + b0_ref[...], 0.0).astype(jnp.bfloat16)
    acc = jnp.dot(xb, wsb_ref[...], preferred_element_type=jnp.float32)
    acc = acc + jnp.dot(net, w1b_ref[...], preferred_element_type=jnp.float32)
    o_ref[...] = acc + b1_ref[...]


@jax.jit
def _run(x, w0, b0, w1, b1, ws):
    n, size_in = x.shape
    size_h = w0.shape[1]
    size_out = w1.shape[1]
    n_steps = n // _TILE_M

    out = pl.pallas_call(
        _block_kernel,
        out_shape=jax.ShapeDtypeStruct((n, size_out), jnp.float32),
        grid=(1, n_steps),
        in_specs=[
            pl.BlockSpec((_TILE_M, size_in), lambda i, j: (j, 0)),
            pl.BlockSpec((size_in, size_h), lambda i, j: (0, 0)),
            pl.BlockSpec((1, size_h), lambda i, j: (0, 0)),
            pl.BlockSpec((size_h, size_out), lambda i, j: (0, 0)),
            pl.BlockSpec((1, size_out), lambda i, j: (0, 0)),
            pl.BlockSpec((size_in, size_out), lambda i, j: (0, 0)),
        ],
        out_specs=pl.BlockSpec((_TILE_M, size_out), lambda i, j: (j, 0)),
        scratch_shapes=[
            pltpu.VMEM((size_in, size_h), jnp.bfloat16),
            pltpu.VMEM((size_h, size_out), jnp.bfloat16),
            pltpu.VMEM((size_in, size_out), jnp.bfloat16),
        ],
        compiler_params=pltpu.CompilerParams(
            dimension_semantics=("parallel", "arbitrary"),
            vmem_limit_bytes=60 * 1024 * 1024,
        ),
        cost_estimate=pl.CostEstimate(
            flops=2 * n * (size_in * size_h + size_h * size_out
                           + size_in * size_out),
            transcendentals=0,
            bytes_accessed=(4 * n * (size_in + size_out)
                            + 4 * (size_in * size_h + size_h * size_out
                                   + size_in * size_out)),
        ),
    )(x, w0, b0, w1, b1, ws)
    return out


def kernel(x, w0, b0, w1, b1, ws):
    return _run(x, w0, b0, w1, b1, ws)
